# initial kernel scaffold (unmeasured)
import jax
import jax.numpy as jnp
from jax import lax
from jax.experimental import pallas as pl
from jax.experimental.pallas import tpu as pltpu

N_DEV = 4
M = 4096
D = 4096
STRIPE = M // N_DEV


def kernel(partial, resid, gamma):
    x = partial.reshape(M, D).astype(jnp.bfloat16)
    gamma2 = gamma.reshape(1, D)

    def body(x_ref, resid_ref, gamma_ref, out_ref,
             sendbuf, comm, xchunk, rchunk, outstage,
             local_sem, store_sem, send_sems, recv_sems):
        my = lax.axis_index("i")
        right = lax.rem(my + 1, N_DEV)
        left = lax.rem(my + N_DEV - 1, N_DEV)

        barrier_sem = pltpu.get_barrier_semaphore()
        for nbr in (left, right):
            pl.semaphore_signal(
                barrier_sem, inc=1,
                device_id=(nbr,), device_id_type=pl.DeviceIdType.MESH,
            )
        pl.semaphore_wait(barrier_sem, 2)

        ld0 = pltpu.make_async_copy(
            x_ref.at[pl.ds(my * STRIPE, STRIPE)], sendbuf, local_sem)
        ld0.start()
        ld0.wait()

        for h in range(N_DEV - 1):
            rc = lax.rem(my - h - 1 + 2 * N_DEV, N_DEV)
            ld = pltpu.make_async_copy(
                x_ref.at[pl.ds(rc * STRIPE, STRIPE)], xchunk, local_sem)
            ld.start()
            slot = h % 2
            rdma = pltpu.make_async_remote_copy(
                src_ref=sendbuf,
                dst_ref=comm.at[slot],
                send_sem=send_sems.at[slot],
                recv_sem=recv_sems.at[slot],
                device_id=(right,),
                device_id_type=pl.DeviceIdType.MESH,
            )
            rdma.start()
            rdma.wait()
            ld.wait()
            sendbuf[...] = comm[slot] + xchunk[...]

        own_c = lax.rem(my + 1, N_DEV)
        ldr = pltpu.make_async_copy(
            resid_ref.at[pl.ds(own_c * STRIPE, STRIPE)], rchunk, local_sem)
        ldr.start()
        ldr.wait()
        y = sendbuf[...].astype(jnp.float32) + rchunk[...]
        rms = jnp.sqrt(jnp.mean(y * y, axis=-1, keepdims=True) + 1e-6)
        normed = y / rms * gamma_ref[...]
        outstage[...] = normed
        sendbuf[...] = normed.astype(jnp.bfloat16)
        st = pltpu.make_async_copy(
            outstage, out_ref.at[pl.ds(own_c * STRIPE, STRIPE)], store_sem)
        st.start()
        st.wait()

        for g in range(N_DEV - 1):
            slot = (g + 1) % 2
            rdma = pltpu.make_async_remote_copy(
                src_ref=sendbuf,
                dst_ref=comm.at[slot],
                send_sem=send_sems.at[slot],
                recv_sem=recv_sems.at[slot],
                device_id=(right,),
                device_id_type=pl.DeviceIdType.MESH,
            )
            rdma.start()
            rdma.wait()
            oc = lax.rem(my - g + 2 * N_DEV, N_DEV)
            sendbuf[...] = comm[slot]
            outstage[...] = comm[slot].astype(jnp.float32)
            st = pltpu.make_async_copy(
                outstage, out_ref.at[pl.ds(oc * STRIPE, STRIPE)], store_sem)
            st.start()
            st.wait()

    return pl.pallas_call(
        body,
        out_shape=jax.ShapeDtypeStruct((M, D), jnp.float32),
        in_specs=[
            pl.BlockSpec(memory_space=pltpu.ANY),
            pl.BlockSpec(memory_space=pltpu.ANY),
            pl.BlockSpec(memory_space=pltpu.VMEM),
        ],
        out_specs=pl.BlockSpec(memory_space=pltpu.ANY),
        scratch_shapes=[
            pltpu.VMEM((STRIPE, D), jnp.bfloat16),
            pltpu.VMEM((2, STRIPE, D), jnp.bfloat16),
            pltpu.VMEM((STRIPE, D), jnp.bfloat16),
            pltpu.VMEM((STRIPE, D), jnp.float32),
            pltpu.VMEM((STRIPE, D), jnp.float32),
            pltpu.SemaphoreType.DMA,
            pltpu.SemaphoreType.DMA,
            pltpu.SemaphoreType.DMA((2,)),
            pltpu.SemaphoreType.DMA((2,)),
        ],
        compiler_params=pltpu.CompilerParams(collective_id=0),
    )(x, resid, gamma2)


# baseline (device time: 669365 ns/iter reference)
import jax
import jax.numpy as jnp
from jax import lax
from jax.experimental import pallas as pl
from jax.experimental.pallas import tpu as pltpu

N_DEV = 4
M = 4096
D = 4096
STRIPE = M // N_DEV


def kernel(partial, resid, gamma):
    x = partial.reshape(M, D).astype(jnp.bfloat16)
    resid_b = resid.astype(jnp.bfloat16)
    gamma2 = gamma.reshape(1, D)

    def body(x_ref, resid_ref, gamma_ref, out_ref,
             sendbuf, comm, xchunk, rchunk,
             local_sem, store_sem, send_sems, recv_sems):
        my = lax.axis_index("i")
        right = lax.rem(my + 1, N_DEV)
        left = lax.rem(my + N_DEV - 1, N_DEV)

        barrier_sem = pltpu.get_barrier_semaphore()
        for nbr in (left, right):
            pl.semaphore_signal(
                barrier_sem, inc=1,
                device_id=(nbr,), device_id_type=pl.DeviceIdType.MESH,
            )
        pl.semaphore_wait(barrier_sem, 2)

        ld0 = pltpu.make_async_copy(
            x_ref.at[pl.ds(my * STRIPE, STRIPE)], sendbuf, local_sem)
        ld0.start()
        ld0.wait()

        for h in range(N_DEV - 1):
            rc = lax.rem(my - h - 1 + 2 * N_DEV, N_DEV)
            ld = pltpu.make_async_copy(
                x_ref.at[pl.ds(rc * STRIPE, STRIPE)], xchunk, local_sem)
            ld.start()
            slot = h % 2
            rdma = pltpu.make_async_remote_copy(
                src_ref=sendbuf,
                dst_ref=comm.at[slot],
                send_sem=send_sems.at[slot],
                recv_sem=recv_sems.at[slot],
                device_id=(right,),
                device_id_type=pl.DeviceIdType.MESH,
            )
            rdma.start()
            rdma.wait()
            ld.wait()
            sendbuf[...] = comm[slot] + xchunk[...]

        own_c = lax.rem(my + 1, N_DEV)
        ldr = pltpu.make_async_copy(
            resid_ref.at[pl.ds(own_c * STRIPE, STRIPE)], rchunk, local_sem)
        ldr.start()
        ldr.wait()
        y = sendbuf[...] + rchunk[...]
        yf = y.astype(jnp.float32)
        rms = jnp.sqrt(jnp.mean(yf * yf, axis=-1, keepdims=True) + 1e-6)
        sendbuf[...] = ((yf / rms) * gamma_ref[...]).astype(jnp.bfloat16)
        st = pltpu.make_async_copy(
            sendbuf, out_ref.at[pl.ds(own_c * STRIPE, STRIPE)], store_sem)
        st.start()
        st.wait()

        for g in range(N_DEV - 1):
            slot = (g + 1) % 2
            rdma = pltpu.make_async_remote_copy(
                src_ref=sendbuf,
                dst_ref=comm.at[slot],
                send_sem=send_sems.at[slot],
                recv_sem=recv_sems.at[slot],
                device_id=(right,),
                device_id_type=pl.DeviceIdType.MESH,
            )
            rdma.start()
            rdma.wait()
            oc = lax.rem(my - g + 2 * N_DEV, N_DEV)
            sendbuf[...] = comm[slot]
            st = pltpu.make_async_copy(
                comm.at[slot], out_ref.at[pl.ds(oc * STRIPE, STRIPE)],
                store_sem)
            st.start()
            st.wait()

    return pl.pallas_call(
        body,
        out_shape=jax.ShapeDtypeStruct((M, D), jnp.bfloat16),
        in_specs=[
            pl.BlockSpec(memory_space=pl.ANY),
            pl.BlockSpec(memory_space=pl.ANY),
            pl.BlockSpec(memory_space=pltpu.VMEM),
        ],
        out_specs=pl.BlockSpec(memory_space=pl.ANY),
        scratch_shapes=[
            pltpu.VMEM((STRIPE, D), jnp.bfloat16),
            pltpu.VMEM((2, STRIPE, D), jnp.bfloat16),
            pltpu.VMEM((STRIPE, D), jnp.bfloat16),
            pltpu.VMEM((STRIPE, D), jnp.bfloat16),
            pltpu.SemaphoreType.DMA,
            pltpu.SemaphoreType.DMA,
            pltpu.SemaphoreType.DMA((2,)),
            pltpu.SemaphoreType.DMA((2,)),
        ],
        compiler_params=pltpu.CompilerParams(
            collective_id=0, vmem_limit_bytes=63 * 1024 * 1024),
    )(x, resid_b, gamma2)


# device time: 340941 ns/iter; 1.9633x vs baseline; 1.9633x over previous
import jax
import jax.numpy as jnp
from jax import lax
from jax.experimental import pallas as pl
from jax.experimental.pallas import tpu as pltpu

N_DEV = 4
M = 4096
D = 4096
STRIPE = M // N_DEV
HALF = D // 2


def kernel(partial, resid, gamma):
    x = partial.reshape(M, D)
    gamma2 = gamma.reshape(1, D)

    def body(x_ref, resid_ref, gamma_ref, out_ref,
             sendbuf, comm_cw, comm_ccw, fchunk,
             ld_sems, store_sem,
             send_cw, recv_cw, send_ccw, recv_ccw):
        my = lax.axis_index("i")
        right = lax.rem(my + 1, N_DEV)
        left = lax.rem(my + N_DEV - 1, N_DEV)

        barrier_sem = pltpu.get_barrier_semaphore()
        for nbr in (left, right):
            pl.semaphore_signal(
                barrier_sem, inc=1,
                device_id=(nbr,), device_id_type=pl.DeviceIdType.MESH,
            )
        pl.semaphore_wait(barrier_sem, 2)

        def load_halves(rc_cw, rc_ccw):
            l = pltpu.make_async_copy(
                x_ref.at[pl.ds(rc_cw * STRIPE, STRIPE), pl.ds(0, HALF)],
                fchunk.at[:, pl.ds(0, HALF)], ld_sems.at[0])
            r = pltpu.make_async_copy(
                x_ref.at[pl.ds(rc_ccw * STRIPE, STRIPE), pl.ds(HALF, HALF)],
                fchunk.at[:, pl.ds(HALF, HALF)], ld_sems.at[1])
            l.start()
            r.start()
            return l, r

        l0, r0 = load_halves(my, lax.rem(my + 2, N_DEV))
        l0.wait()
        r0.wait()
        sendbuf[...] = fchunk[...].astype(jnp.bfloat16)

        for h in range(N_DEV - 1):
            rc_cw = lax.rem(my - h - 1 + 2 * N_DEV, N_DEV)
            rc_ccw = lax.rem(my + h + 3, N_DEV)
            slot = h % 2
            rd_cw = pltpu.make_async_remote_copy(
                src_ref=sendbuf.at[:, pl.ds(0, HALF)],
                dst_ref=comm_cw.at[slot],
                send_sem=send_cw.at[slot], recv_sem=recv_cw.at[slot],
                device_id=(right,), device_id_type=pl.DeviceIdType.MESH)
            rd_ccw = pltpu.make_async_remote_copy(
                src_ref=sendbuf.at[:, pl.ds(HALF, HALF)],
                dst_ref=comm_ccw.at[slot],
                send_sem=send_ccw.at[slot], recv_sem=recv_ccw.at[slot],
                device_id=(left,), device_id_type=pl.DeviceIdType.MESH)
            rd_cw.start()
            rd_ccw.start()
            ld_l, ld_r = load_halves(rc_cw, rc_ccw)
            rd_cw.wait()
            rd_ccw.wait()
            ld_l.wait()
            ld_r.wait()
            sendbuf[:, 0:HALF] = (
                comm_cw[slot] + fchunk[:, 0:HALF].astype(jnp.bfloat16))
            sendbuf[:, HALF:D] = (
                comm_ccw[slot] + fchunk[:, HALF:D].astype(jnp.bfloat16))

        own_c = lax.rem(my + 1, N_DEV)
        ldr = pltpu.make_async_copy(
            resid_ref.at[pl.ds(own_c * STRIPE, STRIPE)], fchunk, ld_sems.at[0])
        ldr.start()
        ldr.wait()
        yf = sendbuf[...].astype(jnp.float32) + fchunk[...]
        rms = jnp.sqrt(jnp.mean(yf * yf, axis=-1, keepdims=True) + 1e-6)
        sendbuf[...] = ((yf / rms) * gamma_ref[...]).astype(jnp.bfloat16)
        st = pltpu.make_async_copy(
            sendbuf, out_ref.at[pl.ds(own_c * STRIPE, STRIPE)], store_sem)
        st.start()
        st.wait()

        for g in range(N_DEV - 1):
            slot = (g + 1) % 2
            prev = g % 2
            if g == 0:
                src_cw = sendbuf.at[:, pl.ds(0, HALF)]
                src_ccw = sendbuf.at[:, pl.ds(HALF, HALF)]
            else:
                src_cw = comm_cw.at[prev]
                src_ccw = comm_ccw.at[prev]
            rd_cw = pltpu.make_async_remote_copy(
                src_ref=src_cw, dst_ref=comm_cw.at[slot],
                send_sem=send_cw.at[slot], recv_sem=recv_cw.at[slot],
                device_id=(right,), device_id_type=pl.DeviceIdType.MESH)
            rd_ccw = pltpu.make_async_remote_copy(
                src_ref=src_ccw, dst_ref=comm_ccw.at[slot],
                send_sem=send_ccw.at[slot], recv_sem=recv_ccw.at[slot],
                device_id=(left,), device_id_type=pl.DeviceIdType.MESH)
            rd_cw.start()
            rd_ccw.start()
            rd_cw.wait()
            rd_ccw.wait()
            oc_cw = lax.rem(my - g + 2 * N_DEV, N_DEV)
            oc_ccw = lax.rem(my + g + 2, N_DEV)
            st_cw = pltpu.make_async_copy(
                comm_cw.at[slot],
                out_ref.at[pl.ds(oc_cw * STRIPE, STRIPE), pl.ds(0, HALF)],
                store_sem)
            st_cw.start()
            st_cw.wait()
            st_ccw = pltpu.make_async_copy(
                comm_ccw.at[slot],
                out_ref.at[pl.ds(oc_ccw * STRIPE, STRIPE), pl.ds(HALF, HALF)],
                store_sem)
            st_ccw.start()
            st_ccw.wait()

    return pl.pallas_call(
        body,
        out_shape=jax.ShapeDtypeStruct((M, D), jnp.bfloat16),
        in_specs=[
            pl.BlockSpec(memory_space=pl.ANY),
            pl.BlockSpec(memory_space=pl.ANY),
            pl.BlockSpec(memory_space=pltpu.VMEM),
        ],
        out_specs=pl.BlockSpec(memory_space=pl.ANY),
        scratch_shapes=[
            pltpu.VMEM((STRIPE, D), jnp.bfloat16),
            pltpu.VMEM((2, STRIPE, HALF), jnp.bfloat16),
            pltpu.VMEM((2, STRIPE, HALF), jnp.bfloat16),
            pltpu.VMEM((STRIPE, D), jnp.float32),
            pltpu.SemaphoreType.DMA((2,)),
            pltpu.SemaphoreType.DMA,
            pltpu.SemaphoreType.DMA((2,)),
            pltpu.SemaphoreType.DMA((2,)),
            pltpu.SemaphoreType.DMA((2,)),
            pltpu.SemaphoreType.DMA((2,)),
        ],
        compiler_params=pltpu.CompilerParams(
            collective_id=0, vmem_limit_bytes=63 * 1024 * 1024),
    )(x, resid, gamma2)


# device time: 313145 ns/iter; 2.1376x vs baseline; 1.0888x over previous
import jax
import jax.numpy as jnp
from jax import lax
from jax.experimental import pallas as pl
from jax.experimental.pallas import tpu as pltpu

N_DEV = 4
M = 4096
D = 4096
STRIPE = M // N_DEV
HALF = D // 2
SEG = 2
SEGR = STRIPE // SEG


def kernel(partial, resid, gamma):
    x = partial.reshape(M, D)
    gamma2 = gamma.reshape(1, D)

    def body(x_ref, resid_ref, gamma_ref, out_ref,
             sendbuf, comm_cw, comm_ccw, fchunk,
             ld_sems, own_store_sem, st_sems,
             send_cw, recv_cw, send_ccw, recv_ccw):
        my = lax.axis_index("i")
        right = lax.rem(my + 1, N_DEV)
        left = lax.rem(my + N_DEV - 1, N_DEV)

        barrier_sem = pltpu.get_barrier_semaphore()
        for nbr in (left, right):
            pl.semaphore_signal(
                barrier_sem, inc=1,
                device_id=(nbr,), device_id_type=pl.DeviceIdType.MESH,
            )
        pl.semaphore_wait(barrier_sem, 2)

        cfg = {
            "cw": (comm_cw, send_cw, recv_cw, 0, right),
            "ccw": (comm_ccw, send_ccw, recv_ccw, HALF, left),
        }

        def sb_seg(dname, seg):
            coff = cfg[dname][3]
            return sendbuf.at[pl.ds(seg * SEGR, SEGR), pl.ds(coff, HALF)]

        def remote(src, dname, slot, seg):
            comm, ssem, rsem, _, dev = cfg[dname]
            return pltpu.make_async_remote_copy(
                src_ref=src,
                dst_ref=comm.at[slot, pl.ds(seg * SEGR, SEGR)],
                send_sem=ssem.at[slot, seg],
                recv_sem=rsem.at[slot, seg],
                device_id=(dev,), device_id_type=pl.DeviceIdType.MESH)

        def load_half(rc, dname, sem):
            coff = cfg[dname][3]
            c = pltpu.make_async_copy(
                x_ref.at[pl.ds(rc * STRIPE, STRIPE), pl.ds(coff, HALF)],
                fchunk.at[:, pl.ds(coff, HALF)], sem)
            c.start()
            return c

        sends = {}

        l0 = load_half(my, "cw", ld_sems.at[0])
        r0 = load_half(lax.rem(my + 2, N_DEV), "ccw", ld_sems.at[1])
        l0.wait()
        sendbuf[:, 0:HALF] = fchunk[:, 0:HALF].astype(jnp.bfloat16)
        for seg in range(SEG):
            rd = remote(sb_seg("cw", seg), "cw", 0, seg)
            rd.start()
            sends[(0, "cw", seg)] = rd
        r0.wait()
        sendbuf[:, HALF:D] = fchunk[:, HALF:D].astype(jnp.bfloat16)
        for seg in range(SEG):
            rd = remote(sb_seg("ccw", seg), "ccw", 0, seg)
            rd.start()
            sends[(0, "ccw", seg)] = rd

        for h in range(N_DEV - 1):
            slot = h % 2
            rc_cw = lax.rem(my - h - 1 + 2 * N_DEV, N_DEV)
            rc_ccw = lax.rem(my + h + 3, N_DEV)
            lds = {"cw": load_half(rc_cw, "cw", ld_sems.at[0]),
                   "ccw": load_half(rc_ccw, "ccw", ld_sems.at[1])}
            waited_ld = {"cw": False, "ccw": False}
            for seg in range(SEG):
                for dname in ("cw", "ccw"):
                    comm, _, _, coff, _ = cfg[dname]
                    rd = sends[(h, dname, seg)]
                    rd.wait_recv()
                    if not waited_ld[dname]:
                        lds[dname].wait()
                        waited_ld[dname] = True
                    rd.wait_send()
                    r0_, r1_ = seg * SEGR, (seg + 1) * SEGR
                    sendbuf[r0_:r1_, coff:coff + HALF] = (
                        comm[slot, r0_:r1_, :]
                        + fchunk[r0_:r1_, coff:coff + HALF].astype(
                            jnp.bfloat16))
                    if h < N_DEV - 2:
                        nrd = remote(sb_seg(dname, seg), dname,
                                     (h + 1) % 2, seg)
                        nrd.start()
                        sends[(h + 1, dname, seg)] = nrd

        own_c = lax.rem(my + 1, N_DEV)
        ldr = []
        for seg in range(SEG):
            c = pltpu.make_async_copy(
                resid_ref.at[pl.ds(own_c * STRIPE + seg * SEGR, SEGR)],
                fchunk.at[pl.ds(seg * SEGR, SEGR)], ld_sems.at[seg])
            c.start()
            ldr.append(c)
        ag = {}
        for seg in range(SEG):
            ldr[seg].wait()
            r0_, r1_ = seg * SEGR, (seg + 1) * SEGR
            yf = sendbuf[r0_:r1_, :].astype(jnp.float32) + fchunk[r0_:r1_, :]
            rms = jnp.sqrt(jnp.mean(yf * yf, axis=-1, keepdims=True) + 1e-6)
            sendbuf[r0_:r1_, :] = (
                (yf / rms) * gamma_ref[...]).astype(jnp.bfloat16)
            for dname in ("cw", "ccw"):
                rd = remote(sb_seg(dname, seg), dname, 1, seg)
                rd.start()
                ag[(0, dname, seg)] = rd
        own_st = pltpu.make_async_copy(
            sendbuf, out_ref.at[pl.ds(own_c * STRIPE, STRIPE)], own_store_sem)
        own_st.start()

        pend_store = {}
        for g in range(N_DEV - 1):
            slot = (g + 1) % 2
            oc = {"cw": lax.rem(my - g + 2 * N_DEV, N_DEV),
                  "ccw": lax.rem(my + g + 2, N_DEV)}
            for seg in range(SEG):
                for dname, di in (("cw", 0), ("ccw", 1)):
                    comm, _, _, coff, _ = cfg[dname]
                    rd = ag[(g, dname, seg)]
                    rd.wait_recv()
                    if g < N_DEV - 2:
                        if g >= 1:
                            ag[(g - 1, dname, seg)].wait_send()
                        nrd = remote(
                            comm.at[slot, pl.ds(seg * SEGR, SEGR)],
                            dname, g % 2, seg)
                        nrd.start()
                        ag[(g + 1, dname, seg)] = nrd
                    prev = pend_store.get((dname, seg))
                    if prev is not None:
                        prev.wait()
                    stc = pltpu.make_async_copy(
                        comm.at[slot, pl.ds(seg * SEGR, SEGR)],
                        out_ref.at[
                            pl.ds(oc[dname] * STRIPE + seg * SEGR, SEGR),
                            pl.ds(coff, HALF)],
                        st_sems.at[di, seg])
                    stc.start()
                    pend_store[(dname, seg)] = stc

        for v in pend_store.values():
            v.wait()
        own_st.wait()
        for seg in range(SEG):
            for dname in ("cw", "ccw"):
                ag[(1, dname, seg)].wait_send()
                ag[(2, dname, seg)].wait_send()

    return pl.pallas_call(
        body,
        out_shape=jax.ShapeDtypeStruct((M, D), jnp.bfloat16),
        in_specs=[
            pl.BlockSpec(memory_space=pl.ANY),
            pl.BlockSpec(memory_space=pl.ANY),
            pl.BlockSpec(memory_space=pltpu.VMEM),
        ],
        out_specs=pl.BlockSpec(memory_space=pl.ANY),
        scratch_shapes=[
            pltpu.VMEM((STRIPE, D), jnp.bfloat16),
            pltpu.VMEM((2, STRIPE, HALF), jnp.bfloat16),
            pltpu.VMEM((2, STRIPE, HALF), jnp.bfloat16),
            pltpu.VMEM((STRIPE, D), jnp.float32),
            pltpu.SemaphoreType.DMA((2,)),
            pltpu.SemaphoreType.DMA,
            pltpu.SemaphoreType.DMA((2, 2)),
            pltpu.SemaphoreType.DMA((2, 2)),
            pltpu.SemaphoreType.DMA((2, 2)),
            pltpu.SemaphoreType.DMA((2, 2)),
            pltpu.SemaphoreType.DMA((2, 2)),
        ],
        compiler_params=pltpu.CompilerParams(
            collective_id=0, vmem_limit_bytes=63 * 1024 * 1024),
    )(x, resid, gamma2)


# device time: 304187 ns/iter; 2.2005x vs baseline; 1.0294x over previous
import jax
import jax.numpy as jnp
from jax import lax
from jax.experimental import pallas as pl
from jax.experimental.pallas import tpu as pltpu

N_DEV = 4
M = 4096
D = 4096
STRIPE = M // N_DEV
HALF = D // 2
SEG = 4
SEGR = STRIPE // SEG
AG_SLOT = (2, 0, 1)


def kernel(partial, resid, gamma):
    x = partial.reshape(M, D)
    gamma2 = gamma.reshape(1, D)

    def body(x_ref, resid_ref, gamma_ref, out_ref,
             sendbuf, comm_cw, comm_ccw, fchunk,
             ld_sems, resid_sems, own_store_sem, st_sems,
             send_cw, recv_cw, send_ccw, recv_ccw,
             ag_send_cw, ag_recv_cw, ag_send_ccw, ag_recv_ccw):
        my = lax.axis_index("i")
        right = lax.rem(my + 1, N_DEV)
        left = lax.rem(my + N_DEV - 1, N_DEV)

        barrier_sem = pltpu.get_barrier_semaphore()
        for nbr in (left, right):
            pl.semaphore_signal(
                barrier_sem, inc=1,
                device_id=(nbr,), device_id_type=pl.DeviceIdType.MESH,
            )
        pl.semaphore_wait(barrier_sem, 2)

        cfg = {
            "cw": (comm_cw, send_cw, recv_cw, ag_send_cw, ag_recv_cw,
                   0, right),
            "ccw": (comm_ccw, send_ccw, recv_ccw, ag_send_ccw, ag_recv_ccw,
                    HALF, left),
        }

        def sb_seg(dname, seg):
            coff = cfg[dname][5]
            return sendbuf.at[pl.ds(seg * SEGR, SEGR), pl.ds(coff, HALF)]

        def rs_remote(dname, slot, seg):
            comm, ssem, rsem = cfg[dname][0], cfg[dname][1], cfg[dname][2]
            dev = cfg[dname][6]
            return pltpu.make_async_remote_copy(
                src_ref=sb_seg(dname, seg),
                dst_ref=comm.at[slot, pl.ds(seg * SEGR, SEGR)],
                send_sem=ssem.at[slot, seg],
                recv_sem=rsem.at[slot, seg],
                device_id=(dev,), device_id_type=pl.DeviceIdType.MESH)

        def ag_remote(src, dname, g, seg):
            comm, ssem, rsem = cfg[dname][0], cfg[dname][3], cfg[dname][4]
            dev = cfg[dname][6]
            return pltpu.make_async_remote_copy(
                src_ref=src,
                dst_ref=comm.at[AG_SLOT[g], pl.ds(seg * SEGR, SEGR)],
                send_sem=ssem.at[g, seg],
                recv_sem=rsem.at[g, seg],
                device_id=(dev,), device_id_type=pl.DeviceIdType.MESH)

        def load_half(rc, dname, sem):
            coff = cfg[dname][5]
            c = pltpu.make_async_copy(
                x_ref.at[pl.ds(rc * STRIPE, STRIPE), pl.ds(coff, HALF)],
                fchunk.at[:, pl.ds(coff, HALF)], sem)
            c.start()
            return c

        sends = {}
        ag = {}
        own_c = lax.rem(my + 1, N_DEV)
        resid_lds = [None] * SEG

        def norm_and_ag0(seg):
            resid_lds[seg].wait()
            r0_, r1_ = seg * SEGR, (seg + 1) * SEGR
            yf = sendbuf[r0_:r1_, :].astype(jnp.float32) + fchunk[r0_:r1_, :]
            rms = jnp.sqrt(jnp.mean(yf * yf, axis=-1, keepdims=True) + 1e-6)
            sendbuf[r0_:r1_, :] = (
                (yf / rms) * gamma_ref[...]).astype(jnp.bfloat16)
            for dname in ("cw", "ccw"):
                rd = ag_remote(sb_seg(dname, seg), dname, 0, seg)
                rd.start()
                ag[(0, dname, seg)] = rd

        l0 = load_half(my, "cw", ld_sems.at[0])
        r0 = load_half(lax.rem(my + 2, N_DEV), "ccw", ld_sems.at[1])
        l0.wait()
        for seg in range(SEG):
            s0, s1 = seg * SEGR, (seg + 1) * SEGR
            sendbuf[s0:s1, 0:HALF] = fchunk[s0:s1, 0:HALF].astype(
                jnp.bfloat16)
            rd = rs_remote("cw", 0, seg)
            rd.start()
            sends[(0, "cw", seg)] = rd
        r0.wait()
        for seg in range(SEG):
            s0, s1 = seg * SEGR, (seg + 1) * SEGR
            sendbuf[s0:s1, HALF:D] = fchunk[s0:s1, HALF:D].astype(
                jnp.bfloat16)
            rd = rs_remote("ccw", 0, seg)
            rd.start()
            sends[(0, "ccw", seg)] = rd

        for h in range(N_DEV - 1):
            slot = h % 2
            rc_cw = lax.rem(my - h - 1 + 2 * N_DEV, N_DEV)
            rc_ccw = lax.rem(my + h + 3, N_DEV)
            lds = {"cw": load_half(rc_cw, "cw", ld_sems.at[0]),
                   "ccw": load_half(rc_ccw, "ccw", ld_sems.at[1])}
            waited_ld = {"cw": False, "ccw": False}
            for seg in range(SEG):
                for dname in ("cw", "ccw"):
                    comm, coff = cfg[dname][0], cfg[dname][5]
                    rd = sends[(h, dname, seg)]
                    rd.wait_recv()
                    if not waited_ld[dname]:
                        lds[dname].wait()
                        waited_ld[dname] = True
                    rd.wait_send()
                    s0, s1 = seg * SEGR, (seg + 1) * SEGR
                    sendbuf[s0:s1, coff:coff + HALF] = (
                        comm[slot, s0:s1, :]
                        + fchunk[s0:s1, coff:coff + HALF].astype(
                            jnp.bfloat16))
                    if h < N_DEV - 2:
                        nrd = rs_remote(dname, (h + 1) % 2, seg)
                        nrd.start()
                        sends[(h + 1, dname, seg)] = nrd
                if h == N_DEV - 2:
                    c = pltpu.make_async_copy(
                        resid_ref.at[
                            pl.ds(own_c * STRIPE + seg * SEGR, SEGR)],
                        fchunk.at[pl.ds(seg * SEGR, SEGR)],
                        resid_sems.at[seg])
                    c.start()
                    resid_lds[seg] = c
                    if seg >= 1:
                        norm_and_ag0(seg - 1)
        norm_and_ag0(SEG - 1)
        own_st = pltpu.make_async_copy(
            sendbuf, out_ref.at[pl.ds(own_c * STRIPE, STRIPE)], own_store_sem)
        own_st.start()

        pend_store = {}
        for g in range(N_DEV - 1):
            slot = AG_SLOT[g]
            oc = {"cw": lax.rem(my - g + 2 * N_DEV, N_DEV),
                  "ccw": lax.rem(my + g + 2, N_DEV)}
            for seg in range(SEG):
                for dname, di in (("cw", 0), ("ccw", 1)):
                    comm, coff = cfg[dname][0], cfg[dname][5]
                    rd = ag[(g, dname, seg)]
                    rd.wait_recv()
                    if g < N_DEV - 2:
                        nrd = ag_remote(
                            comm.at[slot, pl.ds(seg * SEGR, SEGR)],
                            dname, g + 1, seg)
                        nrd.start()
                        ag[(g + 1, dname, seg)] = nrd
                    prev = pend_store.get((dname, seg))
                    if prev is not None:
                        prev.wait()
                    stc = pltpu.make_async_copy(
                        comm.at[slot, pl.ds(seg * SEGR, SEGR)],
                        out_ref.at[
                            pl.ds(oc[dname] * STRIPE + seg * SEGR, SEGR),
                            pl.ds(coff, HALF)],
                        st_sems.at[di, seg])
                    stc.start()
                    pend_store[(dname, seg)] = stc

        for v in pend_store.values():
            v.wait()
        own_st.wait()
        for g in range(N_DEV - 1):
            for seg in range(SEG):
                for dname in ("cw", "ccw"):
                    ag[(g, dname, seg)].wait_send()

    return pl.pallas_call(
        body,
        out_shape=jax.ShapeDtypeStruct((M, D), jnp.bfloat16),
        in_specs=[
            pl.BlockSpec(memory_space=pl.ANY),
            pl.BlockSpec(memory_space=pl.ANY),
            pl.BlockSpec(memory_space=pltpu.VMEM),
        ],
        out_specs=pl.BlockSpec(memory_space=pl.ANY),
        scratch_shapes=[
            pltpu.VMEM((STRIPE, D), jnp.bfloat16),
            pltpu.VMEM((3, STRIPE, HALF), jnp.bfloat16),
            pltpu.VMEM((3, STRIPE, HALF), jnp.bfloat16),
            pltpu.VMEM((STRIPE, D), jnp.float32),
            pltpu.SemaphoreType.DMA((2,)),
            pltpu.SemaphoreType.DMA((SEG,)),
            pltpu.SemaphoreType.DMA,
            pltpu.SemaphoreType.DMA((2, SEG)),
            pltpu.SemaphoreType.DMA((2, SEG)),
            pltpu.SemaphoreType.DMA((2, SEG)),
            pltpu.SemaphoreType.DMA((2, SEG)),
            pltpu.SemaphoreType.DMA((2, SEG)),
            pltpu.SemaphoreType.DMA((3, SEG)),
            pltpu.SemaphoreType.DMA((3, SEG)),
            pltpu.SemaphoreType.DMA((3, SEG)),
            pltpu.SemaphoreType.DMA((3, SEG)),
        ],
        compiler_params=pltpu.CompilerParams(
            collective_id=0, vmem_limit_bytes=63 * 1024 * 1024),
    )(x, resid, gamma2)
